# baseline (device time: 28318 ns/iter reference)
import jax
import jax.numpy as jnp
from jax import lax
from jax.experimental import pallas as pl
from jax.experimental.pallas import tpu as pltpu

N_DEV = 4
N_HOP = N_DEV - 1
N_SUB = 2


def kernel(x):
    _, m, n_total = x.shape
    n_out = n_total // N_DEV
    n_half = n_out // 2
    m_sub = m // N_SUB

    def body(
        x_ref, out_ref,
        comm_r, comm_l, xv_r, xv_l, out_v,
        send_sems_r, recv_sems_r, send_sems_l, recv_sems_l,
        in_sems_r, in_sems_l, out_sems,
    ):
        my = lax.axis_index("i")
        left = (my + N_DEV - 1) % N_DEV
        right = (my + 1) % N_DEV

        barrier_sem = pltpu.get_barrier_semaphore()
        for nbr in [left, right]:
            pl.semaphore_signal(
                barrier_sem, inc=1,
                device_id=(nbr,), device_id_type=pl.DeviceIdType.MESH,
            )
        pl.semaphore_wait(barrier_sem, 2)

        in_copies_r, in_copies_l = [], []
        for k in range(N_DEV):
            c_r = (my + N_DEV - 1 - k) % N_DEV
            c_l = (my + 1 + k) % N_DEV
            rcp = pltpu.make_async_copy(
                x_ref.at[0, :, pl.ds(c_r * n_out, n_half)],
                xv_r.at[k],
                in_sems_r.at[k],
            )
            rcp.start()
            in_copies_r.append(rcp)
            lcp = pltpu.make_async_copy(
                x_ref.at[0, :, pl.ds(c_l * n_out + n_half, n_half)],
                xv_l.at[k],
                in_sems_l.at[k],
            )
            lcp.start()
            in_copies_l.append(lcp)

        def make_rdma(comm, send_sems, recv_sems, h, j, dst):
            return pltpu.make_async_remote_copy(
                src_ref=comm.at[h, pl.ds(j * m_sub, m_sub), :],
                dst_ref=comm.at[h + 1, pl.ds(j * m_sub, m_sub), :],
                send_sem=send_sems.at[h, j],
                recv_sem=recv_sems.at[h + 1, j],
                device_id=(dst,),
                device_id_type=pl.DeviceIdType.MESH,
            )

        in_copies_r[0].wait()
        in_copies_l[0].wait()
        sends = []
        for j in range(N_SUB):
            rows = pl.ds(j * m_sub, m_sub)
            comm_r[0, rows, :] = xv_r[0, rows, :].astype(jnp.bfloat16)
            rd = make_rdma(comm_r, send_sems_r, recv_sems_r, 0, j, right)
            rd.start()
            sends.append(rd)
            comm_l[0, rows, :] = xv_l[0, rows, :].astype(jnp.bfloat16)
            ld = make_rdma(comm_l, send_sems_l, recv_sems_l, 0, j, left)
            ld.start()
            sends.append(ld)

        out_copies = []
        for h in range(1, N_HOP + 1):
            in_copies_r[h].wait()
            in_copies_l[h].wait()
            for j in range(N_SUB):
                rows = pl.ds(j * m_sub, m_sub)
                rrecv = make_rdma(comm_r, send_sems_r, recv_sems_r, h - 1, j, right)
                rrecv.wait_recv()
                if h < N_HOP:
                    comm_r[h, rows, :] = (
                        comm_r[h, rows, :]
                        + xv_r[h, rows, :].astype(jnp.bfloat16)
                    )
                    rd = make_rdma(comm_r, send_sems_r, recv_sems_r, h, j, right)
                    rd.start()
                    sends.append(rd)
                else:
                    out_v[rows, pl.ds(0, n_half)] = (
                        comm_r[h, rows, :].astype(jnp.float32)
                        + xv_r[h, rows, :]
                    )

                lrecv = make_rdma(comm_l, send_sems_l, recv_sems_l, h - 1, j, left)
                lrecv.wait_recv()
                if h < N_HOP:
                    comm_l[h, rows, :] = (
                        comm_l[h, rows, :]
                        + xv_l[h, rows, :].astype(jnp.bfloat16)
                    )
                    ld = make_rdma(comm_l, send_sems_l, recv_sems_l, h, j, left)
                    ld.start()
                    sends.append(ld)
                else:
                    out_v[rows, pl.ds(n_half, n_half)] = (
                        comm_l[h, rows, :].astype(jnp.float32)
                        + xv_l[h, rows, :]
                    )
                    ocp = pltpu.make_async_copy(
                        out_v.at[rows, :], out_ref.at[rows, :], out_sems.at[j]
                    )
                    ocp.start()
                    out_copies.append(ocp)

        for cp in out_copies:
            cp.wait()
        for rd in sends:
            rd.wait_send()

    return pl.pallas_call(
        body,
        out_shape=jax.ShapeDtypeStruct((m, n_out), jnp.float32),
        in_specs=[pl.BlockSpec(memory_space=pl.ANY)],
        out_specs=pl.BlockSpec(memory_space=pl.ANY),
        scratch_shapes=[
            pltpu.VMEM((N_HOP + 1, m, n_half), jnp.bfloat16),
            pltpu.VMEM((N_HOP + 1, m, n_half), jnp.bfloat16),
            pltpu.VMEM((N_DEV, m, n_half), jnp.float32),
            pltpu.VMEM((N_DEV, m, n_half), jnp.float32),
            pltpu.VMEM((m, n_out), jnp.float32),
            pltpu.SemaphoreType.DMA((N_HOP + 1, N_SUB)),
            pltpu.SemaphoreType.DMA((N_HOP + 1, N_SUB)),
            pltpu.SemaphoreType.DMA((N_HOP + 1, N_SUB)),
            pltpu.SemaphoreType.DMA((N_HOP + 1, N_SUB)),
            pltpu.SemaphoreType.DMA((N_DEV,)),
            pltpu.SemaphoreType.DMA((N_DEV,)),
            pltpu.SemaphoreType.DMA((N_SUB,)),
        ],
        compiler_params=pltpu.CompilerParams(collective_id=0),
    )(x)


# device time: 27574 ns/iter; 1.0270x vs baseline; 1.0270x over previous
import jax
import jax.numpy as jnp
from jax import lax
from jax.experimental import pallas as pl
from jax.experimental.pallas import tpu as pltpu

N_DEV = 4
N_HOP = N_DEV - 1
N_SUB = 2


def kernel(x):
    _, m, n_total = x.shape
    n_out = n_total // N_DEV
    n_half = n_out // 2
    m_sub = m // N_SUB

    def body(
        x_ref, out_ref,
        comm_r, comm_l, xv, out_v,
        send_sems_r, recv_sems_r, send_sems_l, recv_sems_l,
        in_sems, out_sems,
    ):
        my = lax.axis_index("i")
        left = (my + N_DEV - 1) % N_DEV
        right = (my + 1) % N_DEV

        barrier_sem = pltpu.get_barrier_semaphore()
        for nbr in [left, right]:
            pl.semaphore_signal(
                barrier_sem, inc=1,
                device_id=(nbr,), device_id_type=pl.DeviceIdType.MESH,
            )
        pl.semaphore_wait(barrier_sem, 2)

        in_copies = []
        for b in range(N_SUB):
            rows = pl.ds(b * m_sub, m_sub)
            cp = pltpu.make_async_copy(
                x_ref.at[0, rows, :], xv.at[rows, :], in_sems.at[b]
            )
            cp.start()
            in_copies.append(cp)

        def lhalf_f32(c, rows):
            return xv[rows, pl.ds(c * n_out, n_half)]

        def rhalf_f32(c, rows):
            return xv[rows, pl.ds(c * n_out + n_half, n_half)]

        def make_rdma(comm, send_sems, recv_sems, h, j, dst):
            return pltpu.make_async_remote_copy(
                src_ref=comm.at[h, pl.ds(j * m_sub, m_sub), :],
                dst_ref=comm.at[h + 1, pl.ds(j * m_sub, m_sub), :],
                send_sem=send_sems.at[h, j],
                recv_sem=recv_sems.at[h + 1, j],
                device_id=(dst,),
                device_id_type=pl.DeviceIdType.MESH,
            )

        c0_r = (my + N_DEV - 1) % N_DEV
        c0_l = (my + 1) % N_DEV
        sends = []
        for j in range(N_SUB):
            rows = pl.ds(j * m_sub, m_sub)
            in_copies[j].wait()
            comm_r[0, rows, :] = lhalf_f32(c0_r, rows).astype(jnp.bfloat16)
            rd = make_rdma(comm_r, send_sems_r, recv_sems_r, 0, j, right)
            rd.start()
            sends.append(rd)
            comm_l[0, rows, :] = rhalf_f32(c0_l, rows).astype(jnp.bfloat16)
            ld = make_rdma(comm_l, send_sems_l, recv_sems_l, 0, j, left)
            ld.start()
            sends.append(ld)

        out_copies = []
        for h in range(1, N_HOP + 1):
            c_r = (my + 2 * N_DEV - 1 - h) % N_DEV
            c_l = (my + 1 + h) % N_DEV
            for j in range(N_SUB):
                rows = pl.ds(j * m_sub, m_sub)
                rrecv = make_rdma(comm_r, send_sems_r, recv_sems_r, h - 1, j, right)
                rrecv.wait_recv()
                if h < N_HOP:
                    comm_r[h, rows, :] = (
                        comm_r[h, rows, :]
                        + lhalf_f32(c_r, rows).astype(jnp.bfloat16)
                    )
                    rd = make_rdma(comm_r, send_sems_r, recv_sems_r, h, j, right)
                    rd.start()
                    sends.append(rd)
                else:
                    out_v[rows, pl.ds(0, n_half)] = (
                        comm_r[h, rows, :].astype(jnp.float32)
                        + lhalf_f32(my, rows)
                    )

                lrecv = make_rdma(comm_l, send_sems_l, recv_sems_l, h - 1, j, left)
                lrecv.wait_recv()
                if h < N_HOP:
                    comm_l[h, rows, :] = (
                        comm_l[h, rows, :]
                        + rhalf_f32(c_l, rows).astype(jnp.bfloat16)
                    )
                    ld = make_rdma(comm_l, send_sems_l, recv_sems_l, h, j, left)
                    ld.start()
                    sends.append(ld)
                else:
                    out_v[rows, pl.ds(n_half, n_half)] = (
                        comm_l[h, rows, :].astype(jnp.float32)
                        + rhalf_f32(my, rows)
                    )
                    ocp = pltpu.make_async_copy(
                        out_v.at[rows, :], out_ref.at[rows, :], out_sems.at[j]
                    )
                    ocp.start()
                    out_copies.append(ocp)

        for cp in out_copies:
            cp.wait()
        for rd in sends:
            rd.wait_send()

    return pl.pallas_call(
        body,
        out_shape=jax.ShapeDtypeStruct((m, n_out), jnp.float32),
        in_specs=[pl.BlockSpec(memory_space=pl.ANY)],
        out_specs=pl.BlockSpec(memory_space=pl.ANY),
        scratch_shapes=[
            pltpu.VMEM((N_HOP + 1, m, n_half), jnp.bfloat16),
            pltpu.VMEM((N_HOP + 1, m, n_half), jnp.bfloat16),
            pltpu.VMEM((m, n_total), jnp.float32),
            pltpu.VMEM((m, n_out), jnp.float32),
            pltpu.SemaphoreType.DMA((N_HOP + 1, N_SUB)),
            pltpu.SemaphoreType.DMA((N_HOP + 1, N_SUB)),
            pltpu.SemaphoreType.DMA((N_HOP + 1, N_SUB)),
            pltpu.SemaphoreType.DMA((N_HOP + 1, N_SUB)),
            pltpu.SemaphoreType.DMA((N_SUB,)),
            pltpu.SemaphoreType.DMA((N_SUB,)),
        ],
        compiler_params=pltpu.CompilerParams(collective_id=0),
    )(x)


# device time: 27483 ns/iter; 1.0304x vs baseline; 1.0033x over previous
import jax
import jax.numpy as jnp
from jax import lax
from jax.experimental import pallas as pl
from jax.experimental.pallas import tpu as pltpu

N_DEV = 4
N_HOP = N_DEV - 1
N_SUB = 2


def kernel(x):
    _, m, n_total = x.shape
    n_out = n_total // N_DEV
    n_half = n_out // 2
    m_sub = m // N_SUB

    def body(
        x_ref, out_ref,
        comm_r, comm_l, xv, out_v,
        send_sems_r, recv_sems_r, send_sems_l, recv_sems_l,
        in_sems, out_sems,
    ):
        my = lax.axis_index("i")
        left = (my + N_DEV - 1) % N_DEV
        right = (my + 1) % N_DEV

        barrier_sem = pltpu.get_barrier_semaphore()
        for nbr in [left, right]:
            pl.semaphore_signal(
                barrier_sem, inc=1,
                device_id=(nbr,), device_id_type=pl.DeviceIdType.MESH,
            )
        pl.semaphore_wait(barrier_sem, 2)

        in_copies = []
        for b in range(N_SUB):
            rows = pl.ds(b * m_sub, m_sub)
            cp = pltpu.make_async_copy(
                x_ref.at[0, rows, :], xv.at[rows, :], in_sems.at[b]
            )
            cp.start()
            in_copies.append(cp)

        def lhalf_f32(c, rows):
            return xv[rows, pl.ds(c * n_out, n_half)]

        def rhalf_f32(c, rows):
            return xv[rows, pl.ds(c * n_out + n_half, n_half)]

        def make_rdma(comm, send_sems, recv_sems, h, j, dst):
            return pltpu.make_async_remote_copy(
                src_ref=comm.at[h, pl.ds(j * m_sub, m_sub), :],
                dst_ref=comm.at[h + 1, pl.ds(j * m_sub, m_sub), :],
                send_sem=send_sems.at[h, j],
                recv_sem=recv_sems.at[h + 1, j],
                device_id=(dst,),
                device_id_type=pl.DeviceIdType.MESH,
            )

        c0_r = (my + N_DEV - 1) % N_DEV
        c0_l = (my + 1) % N_DEV
        sends = []
        for j in range(N_SUB):
            rows = pl.ds(j * m_sub, m_sub)
            in_copies[j].wait()
            comm_r[0, rows, :] = lhalf_f32(c0_r, rows).astype(jnp.bfloat16)
            rd = make_rdma(comm_r, send_sems_r, recv_sems_r, 0, j, right)
            rd.start()
            sends.append(rd)
            comm_l[0, rows, :] = rhalf_f32(c0_l, rows).astype(jnp.bfloat16)
            ld = make_rdma(comm_l, send_sems_l, recv_sems_l, 0, j, left)
            ld.start()
            sends.append(ld)

        out_copies = []
        for h in range(1, N_HOP + 1):
            c_r = (my + 2 * N_DEV - 1 - h) % N_DEV
            c_l = (my + 1 + h) % N_DEV
            for j in range(N_SUB):
                rows = pl.ds(j * m_sub, m_sub)
                rrecv = make_rdma(comm_r, send_sems_r, recv_sems_r, h - 1, j, right)
                rrecv.wait_recv()
                if h < N_HOP:
                    comm_r[h, rows, :] = (
                        comm_r[h, rows, :]
                        + lhalf_f32(c_r, rows).astype(jnp.bfloat16)
                    )
                    rd = make_rdma(comm_r, send_sems_r, recv_sems_r, h, j, right)
                    rd.start()
                    sends.append(rd)
                else:
                    out_v[rows, pl.ds(0, n_half)] = (
                        comm_r[h, rows, :].astype(jnp.float32)
                        + lhalf_f32(my, rows)
                    )

                lrecv = make_rdma(comm_l, send_sems_l, recv_sems_l, h - 1, j, left)
                lrecv.wait_recv()
                if h < N_HOP:
                    comm_l[h, rows, :] = (
                        comm_l[h, rows, :]
                        + rhalf_f32(c_l, rows).astype(jnp.bfloat16)
                    )
                    ld = make_rdma(comm_l, send_sems_l, recv_sems_l, h, j, left)
                    ld.start()
                    sends.append(ld)
                else:
                    out_v[rows, pl.ds(n_half, n_half)] = (
                        comm_l[h, rows, :].astype(jnp.float32)
                        + rhalf_f32(my, rows)
                    )
                    ocp = pltpu.make_async_copy(
                        out_v.at[rows, :], out_ref.at[rows, :], out_sems.at[j]
                    )
                    ocp.start()
                    out_copies.append(ocp)

        for cp in out_copies:
            cp.wait()
        for rd in sends:
            rd.wait_send()

    return pl.pallas_call(
        body,
        out_shape=jax.ShapeDtypeStruct((m, n_out), jnp.float32),
        in_specs=[pl.BlockSpec(memory_space=pltpu.MemorySpace.HBM)],
        out_specs=pl.BlockSpec(memory_space=pltpu.MemorySpace.HBM),
        scratch_shapes=[
            pltpu.VMEM((N_HOP + 1, m, n_half), jnp.bfloat16),
            pltpu.VMEM((N_HOP + 1, m, n_half), jnp.bfloat16),
            pltpu.VMEM((m, n_total), jnp.float32),
            pltpu.VMEM((m, n_out), jnp.float32),
            pltpu.SemaphoreType.DMA((N_HOP + 1, N_SUB)),
            pltpu.SemaphoreType.DMA((N_HOP + 1, N_SUB)),
            pltpu.SemaphoreType.DMA((N_HOP + 1, N_SUB)),
            pltpu.SemaphoreType.DMA((N_HOP + 1, N_SUB)),
            pltpu.SemaphoreType.DMA((N_SUB,)),
            pltpu.SemaphoreType.DMA((N_SUB,)),
        ],
        compiler_params=pltpu.CompilerParams(collective_id=0),
    )(x)


# device time: 26794 ns/iter; 1.0569x vs baseline; 1.0257x over previous
import jax
import jax.numpy as jnp
from jax import lax
from jax.experimental import pallas as pl
from jax.experimental.pallas import tpu as pltpu

N_DEV = 4
N_HOP = N_DEV - 1
N_SUB = 2


def kernel(x):
    _, m, n_total = x.shape
    n_out = n_total // N_DEV
    n_half = n_out // 2
    m_sub = m // N_SUB

    def body(
        x_ref, out_ref,
        comm_r, comm_l, xb,
        send_sems_r, recv_sems_r, send_sems_l, recv_sems_l,
    ):
        my = lax.axis_index("i")
        left = (my + N_DEV - 1) % N_DEV
        right = (my + 1) % N_DEV

        barrier_sem = pltpu.get_barrier_semaphore()
        for nbr in [left, right]:
            pl.semaphore_signal(
                barrier_sem, inc=1,
                device_id=(nbr,), device_id_type=pl.DeviceIdType.MESH,
            )
        pl.semaphore_wait(barrier_sem, 2)

        def make_rdma(comm, send_sems, recv_sems, h, j, dst):
            return pltpu.make_async_remote_copy(
                src_ref=comm.at[h, pl.ds(j * m_sub, m_sub), :],
                dst_ref=comm.at[h + 1, pl.ds(j * m_sub, m_sub), :],
                send_sem=send_sems.at[h, j],
                recv_sem=recv_sems.at[h + 1, j],
                device_id=(dst,),
                device_id_type=pl.DeviceIdType.MESH,
            )

        c0_r = (my + N_DEV - 1) % N_DEV
        c0_l = (my + 1) % N_DEV
        sends = []
        for j in range(N_SUB):
            rows = pl.ds(j * m_sub, m_sub)
            comm_r[0, rows, :] = x_ref[
                0, rows, pl.ds(c0_r * n_out, n_half)
            ].astype(jnp.bfloat16)
            rd = make_rdma(comm_r, send_sems_r, recv_sems_r, 0, j, right)
            rd.start()
            sends.append(rd)
            comm_l[0, rows, :] = x_ref[
                0, rows, pl.ds(c0_l * n_out + n_half, n_half)
            ].astype(jnp.bfloat16)
            ld = make_rdma(comm_l, send_sems_l, recv_sems_l, 0, j, left)
            ld.start()
            sends.append(ld)

        for j in range(N_SUB):
            rows = pl.ds(j * m_sub, m_sub)
            xb[rows, :] = x_ref[0, rows, :].astype(jnp.bfloat16)

        for h in range(1, N_HOP + 1):
            c_r = (my + 2 * N_DEV - 1 - h) % N_DEV
            c_l = (my + 1 + h) % N_DEV
            for j in range(N_SUB):
                rows = pl.ds(j * m_sub, m_sub)
                rrecv = make_rdma(comm_r, send_sems_r, recv_sems_r, h - 1, j, right)
                rrecv.wait_recv()
                if h < N_HOP:
                    comm_r[h, rows, :] = (
                        comm_r[h, rows, :]
                        + xb[rows, pl.ds(c_r * n_out, n_half)]
                    )
                    rd = make_rdma(comm_r, send_sems_r, recv_sems_r, h, j, right)
                    rd.start()
                    sends.append(rd)
                else:
                    out_ref[rows, pl.ds(0, n_half)] = (
                        comm_r[h, rows, :].astype(jnp.float32)
                        + x_ref[0, rows, pl.ds(my * n_out, n_half)]
                    )

                lrecv = make_rdma(comm_l, send_sems_l, recv_sems_l, h - 1, j, left)
                lrecv.wait_recv()
                if h < N_HOP:
                    comm_l[h, rows, :] = (
                        comm_l[h, rows, :]
                        + xb[rows, pl.ds(c_l * n_out + n_half, n_half)]
                    )
                    ld = make_rdma(comm_l, send_sems_l, recv_sems_l, h, j, left)
                    ld.start()
                    sends.append(ld)
                else:
                    out_ref[rows, pl.ds(n_half, n_half)] = (
                        comm_l[h, rows, :].astype(jnp.float32)
                        + x_ref[0, rows, pl.ds(my * n_out + n_half, n_half)]
                    )

        for rd in sends:
            rd.wait_send()

    return pl.pallas_call(
        body,
        out_shape=jax.ShapeDtypeStruct((m, n_out), jnp.float32),
        in_specs=[pl.BlockSpec(memory_space=pltpu.VMEM)],
        out_specs=pl.BlockSpec(memory_space=pltpu.VMEM),
        scratch_shapes=[
            pltpu.VMEM((N_HOP + 1, m, n_half), jnp.bfloat16),
            pltpu.VMEM((N_HOP + 1, m, n_half), jnp.bfloat16),
            pltpu.VMEM((m, n_total), jnp.bfloat16),
            pltpu.SemaphoreType.DMA((N_HOP + 1, N_SUB)),
            pltpu.SemaphoreType.DMA((N_HOP + 1, N_SUB)),
            pltpu.SemaphoreType.DMA((N_HOP + 1, N_SUB)),
            pltpu.SemaphoreType.DMA((N_HOP + 1, N_SUB)),
        ],
        compiler_params=pltpu.CompilerParams(collective_id=0),
    )(x)
